# baseline (speedup 1.0000x reference)
import jax
import jax.numpy as jnp
import numpy as np
from jax.experimental import pallas as pl
from jax.experimental.pallas import tpu as pltpu

_NT = (((1,), (1,)), ((), ()))
_NN = (((1,), (0,)), ((), ()))


_NSUB = 2


def _rvq_body(flat_ref, cbs2_ref, c2_ref, cb123_ref,
              q_ref, codes_ref, loss_ref):
    n_q, bins, _ = cbs2_ref.shape
    R = flat_ref.shape[0]
    ns = _NSUB
    H = R // ns
    iota = jax.lax.broadcasted_iota(jnp.int32, (H, bins), 1)

    resid = [flat_ref[h * H:(h + 1) * H, :] for h in range(ns)]
    acc = [jnp.zeros_like(resid[0]) for _ in range(ns)]
    codes_cols = [[] for _ in range(ns)]
    loss_parts = []
    for q in range(n_q):
        cbs2 = cbs2_ref[q]
        c2 = c2_ref[q]
        stage_loss = []
        for h in range(ns):
            r = resid[h]
            dot2 = jax.lax.dot_general(
                r, cbs2, _NT, preferred_element_type=jnp.float32)
            rsq = jnp.sum(r * r, axis=1, keepdims=True)
            neg = rsq - dot2 + c2
            m = jnp.min(neg, axis=1, keepdims=True)
            idx = jnp.min(jnp.where(neg == m, iota, bins),
                          axis=1, keepdims=True)
            onehot = (iota == idx).astype(jnp.bfloat16)
            oh3 = jnp.concatenate([onehot, onehot, onehot], axis=1)
            quant = jax.lax.dot_general(oh3, cb123_ref[q], _NN,
                                        preferred_element_type=jnp.float32)
            diff = quant - r
            stage_loss.append(jnp.sum(diff * diff, axis=0, keepdims=True))
            qst = r + diff
            resid[h] = r - qst
            acc[h] = acc[h] + qst
            codes_cols[h].append(idx)
        total = stage_loss[0]
        for h in range(1, ns):
            total = total + stage_loss[h]
        loss_parts.append(total)
    for h in range(ns):
        q_ref[h * H:(h + 1) * H, :] = acc[h]
        codes_ref[h * H:(h + 1) * H, :] = jnp.concatenate(codes_cols[h], axis=1)
    loss_ref[0] = jnp.concatenate(loss_parts, axis=0)


def kernel(x, codebooks, sample_rate):
    n_q, bins, D = codebooks.shape
    B, Dx, T = x.shape
    rows = B * T
    CHUNK = 1024
    grid = rows // CHUNK

    flat = x.transpose(0, 2, 1).reshape(rows, D)
    c2 = jnp.sum(codebooks ** 2, axis=-1).reshape(n_q, 1, bins)
    cbs2 = codebooks * 2.0
    def _trunc(v):
        bits = jax.lax.bitcast_convert_type(v, jnp.uint32)
        return jax.lax.bitcast_convert_type(bits & jnp.uint32(0xFFFF0000),
                                            jnp.float32)
    h1 = _trunc(codebooks)
    r1 = codebooks - h1
    h2 = _trunc(r1)
    cb1 = h1.astype(jnp.bfloat16)
    cb2 = h2.astype(jnp.bfloat16)
    cb3 = (r1 - h2).astype(jnp.bfloat16)
    cb123 = jnp.concatenate([cb1, cb2, cb3], axis=1)

    qrows, codes_rows, loss_parts = pl.pallas_call(
        _rvq_body,
        grid=(grid,),
        in_specs=[
            pl.BlockSpec((CHUNK, D), lambda i: (i, 0)),
            pl.BlockSpec((n_q, bins, D), lambda i: (0, 0, 0)),
            pl.BlockSpec((n_q, 1, bins), lambda i: (0, 0, 0)),
            pl.BlockSpec((n_q, 3 * bins, D), lambda i: (0, 0, 0)),
        ],
        out_specs=[
            pl.BlockSpec((CHUNK, D), lambda i: (i, 0)),
            pl.BlockSpec((CHUNK, n_q), lambda i: (i, 0)),
            pl.BlockSpec((1, n_q, D), lambda i: (i, 0, 0)),
        ],
        out_shape=[
            jax.ShapeDtypeStruct((rows, D), jnp.float32),
            jax.ShapeDtypeStruct((rows, n_q), jnp.int32),
            jax.ShapeDtypeStruct((grid, n_q, D), jnp.float32),
        ],
    )(flat, cbs2, c2, cb123)

    quantized_out = qrows.reshape(B, T, D).transpose(0, 2, 1)
    codes = codes_rows.reshape(B, T, n_q).transpose(2, 0, 1)
    losses = loss_parts.sum(axis=(0, 2)) / jnp.float32(rows * D)
    commit_loss = jnp.mean(losses)
    bw_per_q = float(np.log2(bins)) * sample_rate / 1000.0
    bw = jnp.asarray(n_q * bw_per_q, dtype=x.dtype)
    return (quantized_out, codes, bw, commit_loss)


# speedup vs baseline: 1.1077x; 1.1077x over previous
import jax
import jax.numpy as jnp
import numpy as np
from jax.experimental import pallas as pl
from jax.experimental.pallas import tpu as pltpu

_NT = (((1,), (1,)), ((), ()))
_NN = (((1,), (0,)), ((), ()))
_NSUB = 2


def _rvq_body(x_ref, cb_ref, q_ref, codes_ref, loss_ref,
              cbs2_s, c2_s, cb123_s):
    n_q, bins, D = cb_ref.shape
    b = pl.program_id(0)

    @pl.when(b == 0)
    def _prep():
        for q in range(n_q):
            cb = cb_ref[q]
            cbs2_s[q] = cb + cb
            c2col = jnp.sum(cb * cb, axis=1, keepdims=True)
            c2_s[q] = jnp.transpose(c2col, (1, 0))
            bits = jax.lax.bitcast_convert_type(cb, jnp.uint32)
            h1 = jax.lax.bitcast_convert_type(
                bits & jnp.uint32(0xFFFF0000), jnp.float32)
            r1 = cb - h1
            bits1 = jax.lax.bitcast_convert_type(r1, jnp.uint32)
            h2 = jax.lax.bitcast_convert_type(
                bits1 & jnp.uint32(0xFFFF0000), jnp.float32)
            cb123_s[q, :bins, :] = h1.astype(jnp.bfloat16)
            cb123_s[q, bins:2 * bins, :] = h2.astype(jnp.bfloat16)
            cb123_s[q, 2 * bins:, :] = (r1 - h2).astype(jnp.bfloat16)

    rows = jnp.transpose(x_ref[0], (1, 0))
    R = rows.shape[0]
    ns = _NSUB
    H = R // ns
    iota = jax.lax.broadcasted_iota(jnp.int32, (H, bins), 1)

    resid = [rows[h * H:(h + 1) * H, :] for h in range(ns)]
    acc = [jnp.zeros_like(resid[0]) for _ in range(ns)]
    codes_cols = [[] for _ in range(ns)]
    loss_parts = []
    for q in range(n_q):
        cbs2 = cbs2_s[q]
        c2 = c2_s[q]
        stage_loss = []
        for h in range(ns):
            r = resid[h]
            dot2 = jax.lax.dot_general(
                r, cbs2, _NT, preferred_element_type=jnp.float32)
            rsq = jnp.sum(r * r, axis=1, keepdims=True)
            neg = rsq - dot2 + c2
            m = jnp.min(neg, axis=1, keepdims=True)
            idx = jnp.min(jnp.where(neg == m, iota, bins),
                          axis=1, keepdims=True)
            onehot = (iota == idx).astype(jnp.bfloat16)
            oh3 = jnp.concatenate([onehot, onehot, onehot], axis=1)
            quant = jax.lax.dot_general(oh3, cb123_s[q], _NN,
                                        preferred_element_type=jnp.float32)
            diff = quant - r
            stage_loss.append(jnp.sum(diff * diff, axis=0, keepdims=True))
            qst = r + diff
            resid[h] = r - qst
            acc[h] = acc[h] + qst
            codes_cols[h].append(idx)
        total = stage_loss[0]
        for h in range(1, ns):
            total = total + stage_loss[h]
        loss_parts.append(total)
    out_rows = jnp.concatenate(acc, axis=0)
    q_ref[0] = jnp.transpose(out_rows, (1, 0))
    for h in range(ns):
        codes_ref[h * H:(h + 1) * H, :] = jnp.concatenate(codes_cols[h], axis=1)
    loss_ref[0] = jnp.concatenate(loss_parts, axis=0)


def kernel(x, codebooks, sample_rate):
    n_q, bins, D = codebooks.shape
    B, Dx, T = x.shape
    rows = B * T

    qout, codes_rows, loss_parts = pl.pallas_call(
        _rvq_body,
        grid=(B,),
        in_specs=[
            pl.BlockSpec((1, D, T), lambda i: (i, 0, 0)),
            pl.BlockSpec((n_q, bins, D), lambda i: (0, 0, 0)),
        ],
        out_specs=[
            pl.BlockSpec((1, D, T), lambda i: (i, 0, 0)),
            pl.BlockSpec((T, n_q), lambda i: (i, 0)),
            pl.BlockSpec((1, n_q, D), lambda i: (i, 0, 0)),
        ],
        out_shape=[
            jax.ShapeDtypeStruct((B, D, T), jnp.float32),
            jax.ShapeDtypeStruct((rows, n_q), jnp.int32),
            jax.ShapeDtypeStruct((B, n_q, D), jnp.float32),
        ],
        scratch_shapes=[
            pltpu.VMEM((n_q, bins, D), jnp.float32),
            pltpu.VMEM((n_q, 1, bins), jnp.float32),
            pltpu.VMEM((n_q, 3 * bins, D), jnp.bfloat16),
        ],
    )(x, codebooks)

    codes = codes_rows.reshape(B, T, n_q).transpose(2, 0, 1)
    losses = loss_parts.sum(axis=(0, 2)) / jnp.float32(rows * D)
    commit_loss = jnp.mean(losses)
    bw_per_q = float(np.log2(bins)) * sample_rate / 1000.0
    bw = jnp.asarray(n_q * bw_per_q, dtype=x.dtype)
    return (qout, codes, bw, commit_loss)
